# baseline (speedup 1.0000x reference)
import jax
import jax.numpy as jnp
from jax import lax
from jax.experimental import pallas as pl
from jax.experimental.pallas import tpu as pltpu
from jax.experimental.pallas import tpu_sc as plsc

EMBED_DIM = 32
BATCH = 16384
HIST = 200

NC = 2
NS = 16
NW = NC * NS

TOTAL = BATCH * HIST
GROUP_ROWS = 512
NGROUP = TOTAL // GROUP_ROWS
GPW = NGROUP // NW
NPAIR = GPW // 2
BT_PER_H = BATCH // 128
DT = EMBED_DIM // 8

assert GPW * NW == NGROUP and NPAIR * 2 == GPW


def _body(idx_hbm, table_hbm, out_hbm,
          idx0, idx1, rows0, rows1, rt0, rt1,
          si0, si1, sg0, sg1, so0, so1):
    idxv = [idx0, idx1]
    rows = [rows0, rows1]
    rt = [rt0, rt1]
    si = [si0, si1]
    sg = [sg0, sg1]
    so = [so0, so1]

    c = lax.axis_index("c")
    s = lax.axis_index("s")
    g0 = (s * NC + c) * GPW

    iota = lax.iota(jnp.int32, 16)
    row_idx = [iota + (ji * 128 + j16 * 16)
               for ji in range(4) for j16 in range(8)]

    def idx_copy(g, b):
        return pltpu.async_copy(
            idx_hbm.at[pl.ds((g0 + g) * GROUP_ROWS, GROUP_ROWS)],
            idxv[b], si[b])

    def gather(b):
        return pltpu.async_copy(table_hbm.at[idxv[b]], rows[b], sg[b])

    def stores(g, b):
        gg = g0 + g
        h = gg // BT_PER_H
        bt0 = (gg % BT_PER_H) * 4
        for dt in range(DT):
            pltpu.async_copy(rt[b].at[dt], out_hbm.at[h, dt, pl.ds(bt0, 4)],
                             so[b])

    def wait_idx(b):
        pltpu.make_async_copy(
            idx_hbm.at[pl.ds(0, GROUP_ROWS)], idxv[b], si[b]).wait()

    def wait_gather(b):
        pltpu.make_async_copy(table_hbm.at[idxv[b]], rows[b], sg[b]).wait()

    def wait_stores(b):
        for dt in range(DT):
            pltpu.make_async_copy(
                rt[b].at[dt], out_hbm.at[0, 0, pl.ds(0, 4)], so[b]).wait()

    def transpose(b):
        for dt in range(DT):
            cols = [jnp.full((16,), dt * 8 + di, jnp.int32)
                    for di in range(8)]
            for ji in range(4):
                for j16 in range(8):
                    ridx = row_idx[ji * 8 + j16]
                    vs = [plsc.load_gather(rows[b], [ridx, cols[di]])
                          for di in range(8)]
                    for di in range(8):
                        rt[b][dt, ji, di, pl.ds(j16 * 16, 16)] = vs[di]

    idx_copy(0, 0).wait()
    gather(0)
    idx_copy(1, 1)

    def pair_step(p, carry):
        for b in range(2):
            g = 2 * p + b
            wait_gather(b)

            @pl.when(p >= 1)
            def _():
                wait_stores(b)

            transpose(b)
            stores(g, b)

            @pl.when(p < NPAIR - 1)
            def _():
                idx_copy(g + 2, b)

            if b == 0:
                wait_idx(1)
                gather(1)
            else:
                @pl.when(p < NPAIR - 1)
                def _():
                    wait_idx(0)
                    gather(0)
        return carry

    lax.fori_loop(0, NPAIR, pair_step, 0)
    wait_stores(0)
    wait_stores(1)


@jax.jit
def kernel(data, ivectors):
    flat_idx = data.T.reshape(TOTAL)
    mesh = plsc.VectorSubcoreMesh(core_axis_name="c", subcore_axis_name="s")
    out5 = pl.kernel(
        _body,
        out_type=jax.ShapeDtypeStruct(
            (HIST, DT, BATCH // 128, 8, 128), jnp.float32),
        mesh=mesh,
        scratch_types=[
            pltpu.VMEM((GROUP_ROWS,), jnp.int32),
            pltpu.VMEM((GROUP_ROWS,), jnp.int32),
            pltpu.VMEM((GROUP_ROWS, EMBED_DIM), jnp.float32),
            pltpu.VMEM((GROUP_ROWS, EMBED_DIM), jnp.float32),
            pltpu.VMEM((DT, 4, 8, 128), jnp.float32),
            pltpu.VMEM((DT, 4, 8, 128), jnp.float32),
            pltpu.SemaphoreType.DMA,
            pltpu.SemaphoreType.DMA,
            pltpu.SemaphoreType.DMA,
            pltpu.SemaphoreType.DMA,
            pltpu.SemaphoreType.DMA,
            pltpu.SemaphoreType.DMA,
        ],
        compiler_params=pltpu.CompilerParams(
            use_tc_tiling_on_sc=False, needs_layout_passes=False),
    )(flat_idx, ivectors)
    return out5.transpose((2, 4, 0, 1, 3)).reshape(BATCH, HIST, EMBED_DIM)


# speedup vs baseline: 1.0911x; 1.0911x over previous
import jax
import jax.numpy as jnp
from jax import lax
from jax.experimental import pallas as pl
from jax.experimental.pallas import tpu as pltpu
from jax.experimental.pallas import tpu_sc as plsc

EMBED_DIM = 32
BATCH = 16384
HIST = 200

NC = 2
NS = 16
NW = NC * NS

TOTAL = BATCH * HIST
GROUP_ROWS = 512
NGROUP = TOTAL // GROUP_ROWS
GPW = NGROUP // NW
NPAIR = GPW // 2
BT_PER_H = BATCH // 128
DT = EMBED_DIM // 8

assert GPW * NW == NGROUP and NPAIR * 2 == GPW


def _body(idx_hbm, table_hbm, out_hbm,
          idx0, idx1, rows0, rows1, rt0, rt1,
          si0, si1, sg0, sg1, so0, so1):
    idxv = [idx0, idx1]
    rows = [rows0, rows1]
    rt = [rt0, rt1]
    si = [si0, si1]
    sg = [sg0, sg1]
    so = [so0, so1]

    c = lax.axis_index("c")
    s = lax.axis_index("s")
    g0 = (s * NC + c) * GPW

    iota = lax.iota(jnp.int32, 16)
    row_idx = [iota + (ji * 128 + j16 * 16)
               for ji in range(4) for j16 in range(8)]

    def idx_copy(g, b):
        return pltpu.async_copy(
            idx_hbm.at[pl.ds((g0 + g) * GROUP_ROWS, GROUP_ROWS)],
            idxv[b], si[b])

    def gather(b):
        return pltpu.async_copy(table_hbm.at[idxv[b]], rows[b], sg[b])

    def stores(g, b):
        gg = g0 + g
        h = gg // BT_PER_H
        bt0 = (gg % BT_PER_H) * 4
        for dt in range(DT):
            pltpu.async_copy(rt[b].at[dt], out_hbm.at[h, dt, pl.ds(bt0, 4)],
                             so[b])

    def wait_idx(b):
        pltpu.make_async_copy(
            idx_hbm.at[pl.ds(0, GROUP_ROWS)], idxv[b], si[b]).wait()

    def wait_gather(b):
        pltpu.make_async_copy(table_hbm.at[idxv[b]], rows[b], sg[b]).wait()

    def wait_stores(b):
        for dt in range(DT):
            pltpu.make_async_copy(
                rt[b].at[dt], out_hbm.at[0, 0, pl.ds(0, 4)], so[b]).wait()

    def transpose(b):
        for dt in range(DT):
            cols = [jnp.full((16,), dt * 8 + di, jnp.int32)
                    for di in range(8)]
            for ji in range(4):
                for j16 in range(8):
                    ridx = row_idx[ji * 8 + j16]
                    vs = [plsc.load_gather(rows[b], [ridx, cols[di]])
                          for di in range(8)]
                    for di in range(8):
                        rt[b][dt, ji, di, pl.ds(j16 * 16, 16)] = vs[di]

    idx_copy(0, 0).wait()
    gather(0)
    idx_copy(1, 1)

    def pair_step(p, carry):
        for b in range(2):
            g = 2 * p + b
            wait_gather(b)

            if b == 0:
                wait_idx(1)
                gather(1)
            else:
                @pl.when(p < NPAIR - 1)
                def _():
                    wait_idx(0)
                    gather(0)

            @pl.when(p >= 1)
            def _():
                wait_stores(b)

            transpose(b)
            stores(g, b)

            @pl.when(p < NPAIR - 1)
            def _():
                idx_copy(g + 2, b)
        return carry

    lax.fori_loop(0, NPAIR, pair_step, 0)
    wait_stores(0)
    wait_stores(1)


@jax.jit
def kernel(data, ivectors):
    flat_idx = data.T.reshape(TOTAL)
    mesh = plsc.VectorSubcoreMesh(core_axis_name="c", subcore_axis_name="s")
    out5 = pl.kernel(
        _body,
        out_type=jax.ShapeDtypeStruct(
            (HIST, DT, BATCH // 128, 8, 128), jnp.float32),
        mesh=mesh,
        scratch_types=[
            pltpu.VMEM((GROUP_ROWS,), jnp.int32),
            pltpu.VMEM((GROUP_ROWS,), jnp.int32),
            pltpu.VMEM((GROUP_ROWS, EMBED_DIM), jnp.float32),
            pltpu.VMEM((GROUP_ROWS, EMBED_DIM), jnp.float32),
            pltpu.VMEM((DT, 4, 8, 128), jnp.float32),
            pltpu.VMEM((DT, 4, 8, 128), jnp.float32),
            pltpu.SemaphoreType.DMA,
            pltpu.SemaphoreType.DMA,
            pltpu.SemaphoreType.DMA,
            pltpu.SemaphoreType.DMA,
            pltpu.SemaphoreType.DMA,
            pltpu.SemaphoreType.DMA,
        ],
        compiler_params=pltpu.CompilerParams(
            use_tc_tiling_on_sc=False, needs_layout_passes=False),
    )(flat_idx, ivectors)
    return out5.transpose((2, 4, 0, 1, 3)).reshape(BATCH, HIST, EMBED_DIM)
